# baseline (speedup 1.0000x reference)
import functools

import jax
import jax.numpy as jnp
from jax import lax
from jax.experimental import pallas as pl
from jax.experimental.pallas import tpu as pltpu
from jax.experimental.pallas import tpu_sc as plsc

_NPIX = 4 * 512 * 512
_K = int(0.15 * _NPIX)
_KF = float(_K)
_STEPS = 8

_NB = 1152
_BINBASE = 117 << 6
_NW = 32
_PER_W = 2 * _NPIX // _NW
_VCHUNK = 8192


def _loss_body(sem_ref, semlab_ref, cntlog_ref, cntlab_ref, out_ref):
    x = sem_ref[0, :, 0, :, :]
    lab = semlab_ref[0, 0, :, :]
    m = jnp.max(x, axis=0)
    e = jnp.exp(x - m[None, :, :])
    lse = m + jnp.log(jnp.sum(e, axis=0))
    cls = jax.lax.broadcasted_iota(jnp.int32, (19, 1024, 128), 0)
    picked = jnp.sum(jnp.where(cls == lab[None, :, :], x, 0.0), axis=0)
    out_ref[0, :, :] = lse - picked

    z = cntlog_ref[0, 0, :, :]
    y = cntlab_ref[0, 0, :, :]
    out_ref[1, :, :] = (jnp.maximum(z, 0.0) - z * y
                        + jnp.log1p(jnp.exp(-jnp.abs(z))))


def _losses(sem_logits, cnt_logits, sem, cnt):
    sem_r = sem_logits.reshape(4, 19, 2, 1024, 128)
    semlab_r = sem.reshape(4, 2, 1024, 128)
    cntlog_r = cnt_logits.reshape(4, 2, 1024, 128)
    cntlab_r = cnt.reshape(4, 2, 1024, 128)
    return pl.pallas_call(
        _loss_body,
        grid=(_STEPS,),
        in_specs=[
            pl.BlockSpec((1, 19, 1, 1024, 128), lambda g: (g // 2, 0, g % 2, 0, 0)),
            pl.BlockSpec((1, 1, 1024, 128), lambda g: (g // 2, g % 2, 0, 0)),
            pl.BlockSpec((1, 1, 1024, 128), lambda g: (g // 2, g % 2, 0, 0)),
            pl.BlockSpec((1, 1, 1024, 128), lambda g: (g // 2, g % 2, 0, 0)),
        ],
        out_specs=pl.BlockSpec((2, 1024, 128), lambda g: (0, g, 0)),
        out_shape=jax.ShapeDtypeStruct((2, 8192, 128), jnp.float32),
    )(sem_r, semlab_r, cntlog_r, cntlab_r)


@functools.cache
def _make_sc_hist():
    return functools.partial(
        pl.kernel,
        mesh=plsc.VectorSubcoreMesh(core_axis_name="c", subcore_axis_name="s"),
        out_type=jax.ShapeDtypeStruct((_NW, 2, _NB * 16), jnp.float32),
        scratch_types=[
            pltpu.VMEM((_VCHUNK,), jnp.float32),
            pltpu.VMEM((_NB * 16,), jnp.float32),
            pltpu.VMEM((_NB * 16,), jnp.float32),
        ],
        compiler_params=pltpu.CompilerParams(needs_layout_passes=False),
    )(_sc_hist_body)


def _sc_hist_body(losses_hbm, out_hbm, buf, histc, hists):
    wid = lax.axis_index("s") * 2 + lax.axis_index("c")
    base = wid * _PER_W
    zeros = jnp.zeros((16,), jnp.float32)
    ones = jnp.ones((16,), jnp.float32)
    lanes = lax.iota(jnp.int32, 16)

    def zbody(i, carry):
        histc[pl.ds(i * 16, 16)] = zeros
        hists[pl.ds(i * 16, 16)] = zeros
        return carry
    lax.fori_loop(0, _NB, zbody, 0)

    def chunk(ci, carry):
        pltpu.sync_copy(losses_hbm.at[pl.ds(base + ci * _VCHUNK, _VCHUNK)], buf)

        def vec(i, c2):
            x = buf[pl.ds(i * 16, 16)]
            bits = lax.bitcast_convert_type(x, jnp.int32)
            b = lax.shift_right_logical(bits, 17) - _BINBASE
            b = jnp.minimum(jnp.maximum(b, 0), _NB - 1)
            addr = b * 16 + lanes
            plsc.addupdate_scatter(histc, [addr], ones)
            plsc.addupdate_scatter(hists, [addr], x)
            return c2
        lax.fori_loop(0, _VCHUNK // 16, vec, 0)
        return carry
    lax.fori_loop(0, _PER_W // _VCHUNK, chunk, 0)

    pltpu.sync_copy(histc, out_hbm.at[wid, 0])
    pltpu.sync_copy(hists, out_hbm.at[wid, 1])


def _merge_body(hist_ref, out_ref):
    flat = (jax.lax.broadcasted_iota(jnp.int32, (144, 128), 0) * 128
            + jax.lax.broadcasted_iota(jnp.int32, (144, 128), 1))
    binv = jax.lax.shift_right_logical(flat, 4)

    total = jnp.float32(0.0)
    for head in range(2):
        w0 = head * 16
        cnt = jnp.sum(hist_ref[w0:w0 + 16, 0, :, :], axis=0)
        sms = jnp.sum(hist_ref[w0:w0 + 16, 1, :, :], axis=0)

        lob = jnp.int32(0)
        hib = jnp.int32(_NB)
        for _ in range(11):
            mid = lax.div(lob + hib, jnp.int32(2))
            c_mid = jnp.sum(jnp.where(binv >= mid, cnt, 0.0))
            take = c_mid >= _KF
            lob = jnp.where(take, mid, lob)
            hib = jnp.where(take, hib, mid)
        bstar = lob

        sel = binv >= bstar
        above = binv >= (bstar + 1)
        s_sel = jnp.sum(jnp.where(sel, sms, 0.0))
        c_sel = jnp.sum(jnp.where(sel, cnt, 0.0))
        c_above = jnp.sum(jnp.where(above, cnt, 0.0))

        lo_bits = jnp.full((8, 128), (bstar + _BINBASE) << 17, jnp.int32)
        hi_bits = jnp.full((8, 128), (bstar + 1 + _BINBASE) << 17, jnp.int32)
        lo_e = jnp.max(jax.lax.bitcast_convert_type(lo_bits, jnp.float32))
        hi_e = jnp.max(jax.lax.bitcast_convert_type(hi_bits, jnp.float32))

        csub = jnp.maximum(c_sel - c_above, jnp.float32(1.0))
        excess = jnp.maximum(c_sel - _KF, jnp.float32(0.0))
        drop_mean = lo_e + (hi_e - lo_e) * excess / (jnp.float32(2.0) * csub)
        total = total + (s_sel - excess * drop_mean)

    out_ref[0, 0] = total * jnp.float32(1.0 / _K)


def _merge(hist):
    return pl.pallas_call(
        _merge_body,
        grid=(1,),
        in_specs=[pl.BlockSpec((_NW, 2, 144, 128), lambda g: (0, 0, 0, 0))],
        out_specs=pl.BlockSpec(memory_space=pltpu.SMEM),
        out_shape=jax.ShapeDtypeStruct((1, 1), jnp.float32),
    )(hist)


def kernel(sem_logits, cnt_logits, sem, cnt):
    losses = _losses(sem_logits, cnt_logits, sem, cnt)
    flat = losses.reshape(2 * _NPIX)
    hist = _make_sc_hist()(flat)
    hist_r = hist.reshape(_NW, 2, 144, 128)
    return _merge(hist_r)[0, 0]


# speedup vs baseline: 1.0670x; 1.0670x over previous
import functools

import jax
import jax.numpy as jnp
from jax import lax
from jax.experimental import pallas as pl
from jax.experimental.pallas import tpu as pltpu
from jax.experimental.pallas import tpu_sc as plsc

_NPIX = 4 * 512 * 512
_K = int(0.15 * _NPIX)
_KF = float(_K)
_STEPS = 8

_NB = 1152
_BINBASE = 117 << 6
_NW = 32
_PER_W = _NPIX // _NW
_HROWS = _NB * 16 // 128


def _sem_body(sem_ref, semlab_ref, out_ref):
    x = sem_ref[0, :, 0, :, :]
    lab = semlab_ref[0, 0, :, :]
    m = jnp.max(x, axis=0)
    e = jnp.exp(x - m[None, :, :])
    lse = m + jnp.log(jnp.sum(e, axis=0))
    cls = jax.lax.broadcasted_iota(jnp.int32, (19, 1024, 128), 0)
    picked = jnp.sum(jnp.where(cls == lab[None, :, :], x, 0.0), axis=0)
    out_ref[...] = lse - picked


def _sem_losses(sem_logits, sem):
    return pl.pallas_call(
        _sem_body,
        grid=(_STEPS,),
        in_specs=[
            pl.BlockSpec((1, 19, 1, 1024, 128), lambda g: (g // 2, 0, g % 2, 0, 0)),
            pl.BlockSpec((1, 1, 1024, 128), lambda g: (g // 2, g % 2, 0, 0)),
        ],
        out_specs=pl.BlockSpec((1024, 128), lambda g: (g, 0)),
        out_shape=jax.ShapeDtypeStruct((8192, 128), jnp.float32),
    )(sem_logits.reshape(4, 19, 2, 1024, 128), sem.reshape(4, 2, 1024, 128))


def _cnt_body(cntlog_ref, cntlab_ref, out_ref):
    z = cntlog_ref[...]
    y = cntlab_ref[...]
    out_ref[...] = (jnp.maximum(z, 0.0) - z * y
                    + jnp.log1p(jnp.exp(-jnp.abs(z))))


def _cnt_losses(cnt_logits, cnt):
    return pl.pallas_call(
        _cnt_body,
        grid=(1,),
        in_specs=[
            pl.BlockSpec((8192, 128), lambda g: (0, 0)),
            pl.BlockSpec((8192, 128), lambda g: (0, 0)),
        ],
        out_specs=pl.BlockSpec((8192, 128), lambda g: (0, 0)),
        out_shape=jax.ShapeDtypeStruct((8192, 128), jnp.float32),
    )(cnt_logits.reshape(8192, 128), cnt.reshape(8192, 128))


@functools.cache
def _make_sc_hist():
    return functools.partial(
        pl.kernel,
        mesh=plsc.VectorSubcoreMesh(core_axis_name="c", subcore_axis_name="s"),
        out_type=jax.ShapeDtypeStruct((_NW, 2, _NB * 16), jnp.float32),
        scratch_types=[
            pltpu.VMEM((_PER_W,), jnp.float32),
            pltpu.VMEM((_NB * 16,), jnp.float32),
            pltpu.VMEM((_NB * 16,), jnp.float32),
            pltpu.SemaphoreType.DMA,
        ],
        compiler_params=pltpu.CompilerParams(needs_layout_passes=False),
    )(_sc_hist_body)


def _sc_hist_body(losses_hbm, out_hbm, buf, histc, hists, dsem):
    wid = lax.axis_index("s") * 2 + lax.axis_index("c")
    base = wid * _PER_W
    zeros = jnp.zeros((16,), jnp.float32)
    ones = jnp.ones((16,), jnp.float32)
    lanes = lax.iota(jnp.int32, 16)

    cp = pltpu.async_copy(losses_hbm.at[pl.ds(base, _PER_W)], buf, dsem)

    def zbody(i, carry):
        for j in range(4):
            histc[pl.ds(i * 64 + j * 16, 16)] = zeros
            hists[pl.ds(i * 64 + j * 16, 16)] = zeros
        return carry
    lax.fori_loop(0, _NB // 4, zbody, 0)
    cp.wait()

    def vec(i, c2):
        for j in range(4):
            x = buf[pl.ds(i * 64 + j * 16, 16)]
            bits = lax.bitcast_convert_type(x, jnp.int32)
            b = lax.shift_right_logical(bits, 17) - _BINBASE
            b = jnp.minimum(jnp.maximum(b, 0), _NB - 1)
            addr = b * 16 + lanes
            plsc.addupdate_scatter(histc, [addr], ones)
            plsc.addupdate_scatter(hists, [addr], x)
        return c2
    lax.fori_loop(0, _PER_W // 64, vec, 0)

    pltpu.sync_copy(histc, out_hbm.at[wid, 0])
    pltpu.sync_copy(hists, out_hbm.at[wid, 1])


def _merge_body(hs_ref, hc_ref, out_ref):
    flat = (jax.lax.broadcasted_iota(jnp.int32, (_HROWS, 128), 0) * 128
            + jax.lax.broadcasted_iota(jnp.int32, (_HROWS, 128), 1))
    binv = jax.lax.shift_right_logical(flat, 4)

    total = jnp.float32(0.0)
    for ref in (hs_ref, hc_ref):
        cnt = jnp.sum(ref[:, 0, :, :], axis=0)
        sms = jnp.sum(ref[:, 1, :, :], axis=0)

        lob = jnp.int32(0)
        hib = jnp.int32(_NB)
        for _ in range(11):
            mid = lax.div(lob + hib, jnp.int32(2))
            c_mid = jnp.sum(jnp.where(binv >= mid, cnt, 0.0))
            take = c_mid >= _KF
            lob = jnp.where(take, mid, lob)
            hib = jnp.where(take, hib, mid)
        bstar = lob

        sel = binv >= bstar
        above = binv >= (bstar + 1)
        s_sel = jnp.sum(jnp.where(sel, sms, 0.0))
        c_sel = jnp.sum(jnp.where(sel, cnt, 0.0))
        c_above = jnp.sum(jnp.where(above, cnt, 0.0))

        lo_bits = jnp.full((8, 128), (bstar + _BINBASE) << 17, jnp.int32)
        hi_bits = jnp.full((8, 128), (bstar + 1 + _BINBASE) << 17, jnp.int32)
        lo_e = jnp.max(jax.lax.bitcast_convert_type(lo_bits, jnp.float32))
        hi_e = jnp.max(jax.lax.bitcast_convert_type(hi_bits, jnp.float32))

        csub = jnp.maximum(c_sel - c_above, jnp.float32(1.0))
        excess = jnp.maximum(c_sel - _KF, jnp.float32(0.0))
        drop_mean = lo_e + (hi_e - lo_e) * excess / (jnp.float32(2.0) * csub)
        total = total + (s_sel - excess * drop_mean)

    out_ref[0, 0] = total * jnp.float32(1.0 / _K)


def _merge(hs, hc):
    spec = pl.BlockSpec((_NW, 2, _HROWS, 128), lambda g: (0, 0, 0, 0))
    return pl.pallas_call(
        _merge_body,
        grid=(1,),
        in_specs=[spec, spec],
        out_specs=pl.BlockSpec(memory_space=pltpu.SMEM),
        out_shape=jax.ShapeDtypeStruct((1, 1), jnp.float32),
    )(hs, hc)


def kernel(sem_logits, cnt_logits, sem, cnt):
    lc = _cnt_losses(cnt_logits, cnt)
    hc = _make_sc_hist()(lc.reshape(_NPIX))
    ls = _sem_losses(sem_logits, sem)
    hs = _make_sc_hist()(ls.reshape(_NPIX))
    hs_r = hs.reshape(_NW, 2, _HROWS, 128)
    hc_r = hc.reshape(_NW, 2, _HROWS, 128)
    return _merge(hs_r, hc_r)[0, 0]


# speedup vs baseline: 1.0690x; 1.0019x over previous
import functools

import jax
import jax.numpy as jnp
from jax import lax
from jax.experimental import pallas as pl
from jax.experimental.pallas import tpu as pltpu
from jax.experimental.pallas import tpu_sc as plsc

_NPIX = 4 * 512 * 512
_K = int(0.15 * _NPIX)
_KF = float(_K)
_STEPS = 8

_NB = 1152
_BINBASE = 117 << 6
_NW = 32
_PER_W = _NPIX // _NW
_HROWS = _NB * 16 // 128


def _sem_body(sem_ref, semlab_ref, out_ref):
    x = sem_ref[0, :, 0, :, :]
    lab = semlab_ref[0, 0, :, :]
    m = jnp.max(x, axis=0)
    e = jnp.exp(x - m[None, :, :])
    lse = m + jnp.log(jnp.sum(e, axis=0))
    cls = jax.lax.broadcasted_iota(jnp.int32, (19, 1024, 128), 0)
    picked = jnp.sum(jnp.where(cls == lab[None, :, :], x, 0.0), axis=0)
    out_ref[...] = lse - picked


def _sem_losses(sem_logits, sem):
    return pl.pallas_call(
        _sem_body,
        grid=(_STEPS,),
        in_specs=[
            pl.BlockSpec((1, 19, 1, 1024, 128), lambda g: (g // 2, 0, g % 2, 0, 0)),
            pl.BlockSpec((1, 1, 1024, 128), lambda g: (g // 2, g % 2, 0, 0)),
        ],
        out_specs=pl.BlockSpec((1024, 128), lambda g: (g, 0)),
        out_shape=jax.ShapeDtypeStruct((8192, 128), jnp.float32),
    )(sem_logits.reshape(4, 19, 2, 1024, 128), sem.reshape(4, 2, 1024, 128))


def _cnt_body(cntlog_ref, cntlab_ref, out_ref):
    z = cntlog_ref[...]
    y = cntlab_ref[...]
    out_ref[...] = (jnp.maximum(z, 0.0) - z * y
                    + jnp.log1p(jnp.exp(-jnp.abs(z))))


def _cnt_losses(cnt_logits, cnt):
    return pl.pallas_call(
        _cnt_body,
        grid=(1,),
        in_specs=[
            pl.BlockSpec((8192, 128), lambda g: (0, 0)),
            pl.BlockSpec((8192, 128), lambda g: (0, 0)),
        ],
        out_specs=pl.BlockSpec((8192, 128), lambda g: (0, 0)),
        out_shape=jax.ShapeDtypeStruct((8192, 128), jnp.float32),
    )(cnt_logits.reshape(8192, 128), cnt.reshape(8192, 128))


@functools.cache
def _make_sc_hist():
    return functools.partial(
        pl.kernel,
        mesh=plsc.VectorSubcoreMesh(core_axis_name="c", subcore_axis_name="s"),
        out_type=jax.ShapeDtypeStruct((_NW, 2, _NB * 16), jnp.float32),
        scratch_types=[
            pltpu.VMEM((_PER_W,), jnp.float32),
            pltpu.VMEM((_NB * 16,), jnp.float32),
            pltpu.VMEM((_NB * 16,), jnp.float32),
            pltpu.SemaphoreType.DMA,
        ],
        compiler_params=pltpu.CompilerParams(needs_layout_passes=False),
    )(_sc_hist_body)


def _sc_hist_body(losses_hbm, out_hbm, buf, histc, hists, dsem):
    wid = lax.axis_index("s") * 2 + lax.axis_index("c")
    base = wid * _PER_W
    zeros = jnp.zeros((16,), jnp.float32)
    ones = jnp.ones((16,), jnp.float32)
    lanes = lax.iota(jnp.int32, 16)

    cp = pltpu.async_copy(losses_hbm.at[pl.ds(base, _PER_W)], buf, dsem)

    def zbody(i, carry):
        for j in range(4):
            histc[pl.ds(i * 64 + j * 16, 16)] = zeros
            hists[pl.ds(i * 64 + j * 16, 16)] = zeros
        return carry
    lax.fori_loop(0, _NB // 4, zbody, 0)
    cp.wait()

    def vec(i, c2):
        for j in range(8):
            x = buf[pl.ds(i * 128 + j * 16, 16)]
            bits = lax.bitcast_convert_type(x, jnp.int32)
            b = lax.shift_right_logical(bits, 17) - _BINBASE
            b = jnp.minimum(jnp.maximum(b, 0), _NB - 1)
            addr = b * 16 + lanes
            plsc.addupdate_scatter(histc, [addr], ones)
            plsc.addupdate_scatter(hists, [addr], x)
        return c2
    lax.fori_loop(0, _PER_W // 128, vec, 0)

    pltpu.sync_copy(histc, out_hbm.at[wid, 0])
    pltpu.sync_copy(hists, out_hbm.at[wid, 1])


def _merge_body(hs_ref, hc_ref, out_ref):
    flat = (jax.lax.broadcasted_iota(jnp.int32, (_HROWS, 128), 0) * 128
            + jax.lax.broadcasted_iota(jnp.int32, (_HROWS, 128), 1))
    binv = jax.lax.shift_right_logical(flat, 4)

    total = jnp.float32(0.0)
    for ref in (hs_ref, hc_ref):
        cnt = jnp.sum(ref[:, 0, :, :], axis=0)
        sms = jnp.sum(ref[:, 1, :, :], axis=0)

        lob = jnp.int32(0)
        hib = jnp.int32(_NB)
        for _ in range(11):
            mid = lax.div(lob + hib, jnp.int32(2))
            c_mid = jnp.sum(jnp.where(binv >= mid, cnt, 0.0))
            take = c_mid >= _KF
            lob = jnp.where(take, mid, lob)
            hib = jnp.where(take, hib, mid)
        bstar = lob

        sel = binv >= bstar
        above = binv >= (bstar + 1)
        s_sel = jnp.sum(jnp.where(sel, sms, 0.0))
        c_sel = jnp.sum(jnp.where(sel, cnt, 0.0))
        c_above = jnp.sum(jnp.where(above, cnt, 0.0))

        lo_bits = jnp.full((8, 128), (bstar + _BINBASE) << 17, jnp.int32)
        hi_bits = jnp.full((8, 128), (bstar + 1 + _BINBASE) << 17, jnp.int32)
        lo_e = jnp.max(jax.lax.bitcast_convert_type(lo_bits, jnp.float32))
        hi_e = jnp.max(jax.lax.bitcast_convert_type(hi_bits, jnp.float32))

        csub = jnp.maximum(c_sel - c_above, jnp.float32(1.0))
        excess = jnp.maximum(c_sel - _KF, jnp.float32(0.0))
        drop_mean = lo_e + (hi_e - lo_e) * excess / (jnp.float32(2.0) * csub)
        total = total + (s_sel - excess * drop_mean)

    out_ref[0, 0] = total * jnp.float32(1.0 / _K)


def _merge(hs, hc):
    spec = pl.BlockSpec((_NW, 2, _HROWS, 128), lambda g: (0, 0, 0, 0))
    return pl.pallas_call(
        _merge_body,
        grid=(1,),
        in_specs=[spec, spec],
        out_specs=pl.BlockSpec(memory_space=pltpu.SMEM),
        out_shape=jax.ShapeDtypeStruct((1, 1), jnp.float32),
    )(hs, hc)


def kernel(sem_logits, cnt_logits, sem, cnt):
    lc = _cnt_losses(cnt_logits, cnt)
    hc = _make_sc_hist()(lc.reshape(_NPIX))
    ls = _sem_losses(sem_logits, sem)
    hs = _make_sc_hist()(ls.reshape(_NPIX))
    hs_r = hs.reshape(_NW, 2, _HROWS, 128)
    hc_r = hc.reshape(_NW, 2, _HROWS, 128)
    return _merge(hs_r, hc_r)[0, 0]
